# Initial kernel scaffold; baseline (speedup 1.0000x reference)
import jax
import jax.numpy as jnp
from jax.experimental import pallas as pl


def kernel(x, table):
    raise NotImplementedError("write your pallas kernel here")


# speedup vs baseline: 1.1061x; 1.1061x over previous
import functools

import jax
import jax.numpy as jnp
from jax import lax
from jax.experimental import pallas as pl
from jax.experimental.pallas import tpu as pltpu
from jax.experimental.pallas import tpu_sc as plsc

VOCAB = 1000000
EMBED_DIM = 32
BATCH = 16384
FIELDS = 100

NUM_CORES = 2
NUM_SUBCORES = 16
NUM_WORKERS = NUM_CORES * NUM_SUBCORES

TOTAL = BATCH * FIELDS
PER_WORKER = TOTAL // NUM_WORKERS
CHUNK = 1600
NUM_CHUNKS = PER_WORKER // CHUNK


def _emb_body(x_hbm, table_hbm, out_hbm, idx_v, rows_v, sem):
    wid = lax.axis_index("s") * NUM_CORES + lax.axis_index("c")
    base = wid * PER_WORKER

    @pl.loop(0, NUM_CHUNKS)
    def _chunk(t):
        off = base + t * CHUNK
        pltpu.sync_copy(x_hbm.at[pl.ds(off, CHUNK)], idx_v)
        pltpu.async_copy(table_hbm.at[idx_v], rows_v, sem).wait()
        pltpu.sync_copy(rows_v, out_hbm.at[pl.ds(off, CHUNK)])


@jax.jit
def _embed(x_flat, table):
    mesh = plsc.VectorSubcoreMesh(core_axis_name="c", subcore_axis_name="s")
    return pl.kernel(
        _emb_body,
        out_type=jax.ShapeDtypeStruct((TOTAL, EMBED_DIM), jnp.float32),
        mesh=mesh,
        scratch_types=[
            pltpu.VMEM((CHUNK,), jnp.int32),
            pltpu.VMEM((CHUNK, EMBED_DIM), jnp.float32),
            pltpu.SemaphoreType.DMA,
        ],
        compiler_params=pltpu.CompilerParams(use_tc_tiling_on_sc=False),
    )(x_flat, table)


def kernel(x, table):
    out = _embed(x.reshape(-1), table)
    return out.reshape(BATCH, FIELDS, EMBED_DIM)


# speedup vs baseline: 1.1125x; 1.0058x over previous
import functools

import jax
import jax.numpy as jnp
from jax import lax
from jax.experimental import pallas as pl
from jax.experimental.pallas import tpu as pltpu
from jax.experimental.pallas import tpu_sc as plsc

VOCAB = 1000000
EMBED_DIM = 32
BATCH = 16384
FIELDS = 100

NUM_CORES = 2
NUM_SUBCORES = 16
NUM_WORKERS = NUM_CORES * NUM_SUBCORES

TOTAL = BATCH * FIELDS
PER_WORKER = TOTAL // NUM_WORKERS
CHUNK = 1600
NUM_CHUNKS = PER_WORKER // CHUNK


def _emb_body(x_hbm, table_hbm, out_hbm,
              idx0, idx1, rows0, rows1, sg0, sg1, ss0, ss1):
    wid = lax.axis_index("s") * NUM_CORES + lax.axis_index("c")
    base = wid * PER_WORKER
    idx = [idx0, idx1]
    rows = [rows0, rows1]
    sg = [sg0, sg1]
    ss = [ss0, ss1]

    def load_idx(g, b):
        pltpu.sync_copy(x_hbm.at[pl.ds(base + g * CHUNK, CHUNK)], idx[b])

    def start_gather(b):
        pltpu.async_copy(table_hbm.at[idx[b]], rows[b], sg[b])

    def wait_gather(b):
        pltpu.make_async_copy(table_hbm.at[idx[b]], rows[b], sg[b]).wait()

    def start_store(g, b):
        pltpu.async_copy(rows[b], out_hbm.at[pl.ds(base + g * CHUNK, CHUNK)],
                         ss[b])

    def wait_store(g, b):
        pltpu.make_async_copy(rows[b],
                              out_hbm.at[pl.ds(base + g * CHUNK, CHUNK)],
                              ss[b]).wait()

    load_idx(0, 0)
    start_gather(0)
    load_idx(1, 1)
    start_gather(1)

    @pl.loop(0, NUM_CHUNKS - 2, step=2)
    def _pair(g):
        for j in range(2):
            b = j
            wait_gather(b)
            start_store(g + j, b)
            load_idx(g + j + 2, b)
            wait_store(g + j, b)
            start_gather(b)

    for j in range(2):
        g = NUM_CHUNKS - 2 + j
        wait_gather(j)
        start_store(g, j)
    for j in range(2):
        wait_store(NUM_CHUNKS - 2 + j, j)


@jax.jit
def _embed(x_flat, table):
    mesh = plsc.VectorSubcoreMesh(core_axis_name="c", subcore_axis_name="s")
    return pl.kernel(
        _emb_body,
        out_type=jax.ShapeDtypeStruct((TOTAL, EMBED_DIM), jnp.float32),
        mesh=mesh,
        scratch_types=[
            pltpu.VMEM((CHUNK,), jnp.int32),
            pltpu.VMEM((CHUNK,), jnp.int32),
            pltpu.VMEM((CHUNK, EMBED_DIM), jnp.float32),
            pltpu.VMEM((CHUNK, EMBED_DIM), jnp.float32),
            pltpu.SemaphoreType.DMA,
            pltpu.SemaphoreType.DMA,
            pltpu.SemaphoreType.DMA,
            pltpu.SemaphoreType.DMA,
        ],
        compiler_params=pltpu.CompilerParams(use_tc_tiling_on_sc=False),
    )(x_flat, table)


def kernel(x, table):
    out = _embed(x.reshape(-1), table)
    return out.reshape(BATCH, FIELDS, EMBED_DIM)


# speedup vs baseline: 2.0176x; 1.8136x over previous
import functools

import jax
import jax.numpy as jnp
from jax import lax
from jax.experimental import pallas as pl
from jax.experimental.pallas import tpu as pltpu
from jax.experimental.pallas import tpu_sc as plsc

VOCAB = 1000000
EMBED_DIM = 32
BATCH = 16384
FIELDS = 100

NUM_CORES = 2
NUM_SUBCORES = 16
NUM_WORKERS = NUM_CORES * NUM_SUBCORES

TOTAL = BATCH * FIELDS
PER_WORKER = TOTAL // NUM_WORKERS
CHUNK = 320
NUM_CHUNKS = PER_WORKER // CHUNK
WIDE_ROWS = VOCAB // 4
OUT_WROWS = TOTAL // 4


def _emb_body(x_hbm, table_hbm, out_hbm,
              idx0, idx1, widx0, widx1, wide0, wide1, outw0, outw1,
              sg0, sg1, ss0, ss1):
    wid = lax.axis_index("s") * NUM_CORES + lax.axis_index("c")
    base = wid * PER_WORKER
    idx = [idx0, idx1]
    widx = [widx0, widx1]
    wide = [wide0, wide1]
    outw = [outw0, outw1]
    sg = [sg0, sg1]
    ss = [ss0, ss1]

    iota = lax.iota(jnp.int32, 16)

    def prep(g, b):
        pltpu.sync_copy(x_hbm.at[pl.ds(base + g * CHUNK, CHUNK)], idx[b])

        @pl.loop(0, CHUNK // 16)
        def _w(i):
            v = idx[b][pl.ds(i * 16, 16)]
            widx[b][pl.ds(i * 16, 16)] = lax.shift_right_logical(v, 2)

    def start_gather(b):
        pltpu.async_copy(table_hbm.at[widx[b]], wide[b], sg[b])

    def wait_gather(b):
        pltpu.make_async_copy(table_hbm.at[widx[b]], wide[b], sg[b]).wait()

    def extract(b):
        @pl.loop(0, CHUNK // 16)
        def _blk(i):
            j0 = i * 16
            rsel = iota + j0
            s = lax.bitwise_and(idx[b][pl.ds(j0, 16)], 3) * 32
            orow = lax.shift_right_logical(rsel, 2)
            ocol = lax.bitwise_and(rsel, 3) * 32
            for c in range(EMBED_DIM):
                vals = plsc.load_gather(wide[b], [rsel, s + c])
                plsc.store_scatter(outw[b], [orow, ocol + c], vals)

    def start_store(g, b):
        off = pl.multiple_of((base + g * CHUNK) // 4, 8)
        pltpu.async_copy(outw[b], out_hbm.at[pl.ds(off, CHUNK // 4)], ss[b])

    def wait_store(g, b):
        off = pl.multiple_of((base + g * CHUNK) // 4, 8)
        pltpu.make_async_copy(outw[b], out_hbm.at[pl.ds(off, CHUNK // 4)],
                              ss[b]).wait()

    prep(0, 0)
    start_gather(0)
    prep(1, 1)
    start_gather(1)

    @pl.loop(0, NUM_CHUNKS - 2, step=2)
    def _pair(g):
        for j in range(2):
            b = j
            wait_gather(b)
            extract(b)
            start_store(g + j, b)
            prep(g + j + 2, b)
            wait_store(g + j, b)
            start_gather(b)

    for j in range(2):
        g = NUM_CHUNKS - 2 + j
        wait_gather(j)
        extract(j)
        start_store(g, j)
    for j in range(2):
        wait_store(NUM_CHUNKS - 2 + j, j)


@jax.jit
def _embed(x_flat, table_wide):
    mesh = plsc.VectorSubcoreMesh(core_axis_name="c", subcore_axis_name="s")
    return pl.kernel(
        _emb_body,
        out_type=jax.ShapeDtypeStruct((OUT_WROWS, 128), jnp.float32),
        mesh=mesh,
        scratch_types=[
            pltpu.VMEM((CHUNK,), jnp.int32),
            pltpu.VMEM((CHUNK,), jnp.int32),
            pltpu.VMEM((CHUNK,), jnp.int32),
            pltpu.VMEM((CHUNK,), jnp.int32),
            pltpu.VMEM((CHUNK, 128), jnp.float32),
            pltpu.VMEM((CHUNK, 128), jnp.float32),
            pltpu.VMEM((CHUNK // 4, 128), jnp.float32),
            pltpu.VMEM((CHUNK // 4, 128), jnp.float32),
            pltpu.SemaphoreType.DMA,
            pltpu.SemaphoreType.DMA,
            pltpu.SemaphoreType.DMA,
            pltpu.SemaphoreType.DMA,
        ],
        compiler_params=pltpu.CompilerParams(use_tc_tiling_on_sc=True,
                                             needs_layout_passes=False),
    )(x_flat, table_wide)


def kernel(x, table):
    out = _embed(x.reshape(-1), table.reshape(WIDE_ROWS, 128))
    return out.reshape(BATCH, FIELDS, EMBED_DIM)


# speedup vs baseline: 3.5860x; 1.7774x over previous
import functools

import jax
import jax.numpy as jnp
from jax import lax
from jax.experimental import pallas as pl
from jax.experimental.pallas import tpu as pltpu
from jax.experimental.pallas import tpu_sc as plsc

VOCAB = 1000000
EMBED_DIM = 32
BATCH = 16384
FIELDS = 100

NUM_CORES = 2
NUM_SUBCORES = 16
NUM_WORKERS = NUM_CORES * NUM_SUBCORES

TOTAL = BATCH * FIELDS
PER_WORKER = TOTAL // NUM_WORKERS
CHUNK = 320
NUM_CHUNKS = PER_WORKER // CHUNK
WIDE_ROWS = VOCAB // 4
OUT_WROWS = TOTAL // 4


def _emb_body(x_hbm, table_hbm, out_hbm,
              idx0, idx1, widx0, widx1, wide0, wide1, outw0, outw1,
              sg0, sg1, ss0, ss1):
    wid = lax.axis_index("s") * NUM_CORES + lax.axis_index("c")
    base = wid * PER_WORKER
    idx = [idx0, idx1]
    widx = [widx0, widx1]
    wide = [wide0, wide1]
    outw = [outw0, outw1]
    sg = [sg0, sg1]
    ss = [ss0, ss1]

    iota = lax.iota(jnp.int32, 16)

    def prep(g, b):
        pltpu.sync_copy(x_hbm.at[pl.ds(base + g * CHUNK, CHUNK)], idx[b])

        @pl.loop(0, CHUNK // 16)
        def _w(i):
            v = idx[b][pl.ds(i * 16, 16)]
            widx[b][pl.ds(i * 16, 16)] = lax.shift_right_logical(v, 2)

    def start_gather(b):
        pltpu.async_copy(table_hbm.at[widx[b]], wide[b], sg[b])

    def wait_gather(b):
        pltpu.make_async_copy(table_hbm.at[widx[b]], wide[b], sg[b]).wait()

    def extract(b):
        @pl.loop(0, CHUNK // 16)
        def _blk(i):
            j0 = i * 16
            sv = lax.bitwise_and(idx[b][pl.ds(j0, 16)], 3) * 32
            orow0 = lax.shift_right_logical(j0, 4) * 4
            for l in range(16):
                s = sv[l]
                orow = orow0 + l // 4
                ocol = (l % 4) * 32
                for h in range(2):
                    outw[b][orow, pl.ds(ocol + h * 16, 16)] = (
                        wide[b][j0 + l, pl.ds(s + h * 16, 16)])

    def start_store(g, b):
        off = pl.multiple_of((base + g * CHUNK) // 4, 8)
        pltpu.async_copy(outw[b], out_hbm.at[pl.ds(off, CHUNK // 4)], ss[b])

    def wait_store(g, b):
        off = pl.multiple_of((base + g * CHUNK) // 4, 8)
        pltpu.make_async_copy(outw[b], out_hbm.at[pl.ds(off, CHUNK // 4)],
                              ss[b]).wait()

    prep(0, 0)
    start_gather(0)
    prep(1, 1)
    start_gather(1)

    @pl.loop(0, NUM_CHUNKS - 2, step=2)
    def _pair(g):
        for j in range(2):
            b = j
            wait_gather(b)
            extract(b)
            start_store(g + j, b)
            prep(g + j + 2, b)
            wait_store(g + j, b)
            start_gather(b)

    for j in range(2):
        g = NUM_CHUNKS - 2 + j
        wait_gather(j)
        extract(j)
        start_store(g, j)
    for j in range(2):
        wait_store(NUM_CHUNKS - 2 + j, j)


@jax.jit
def _embed(x_flat, table_wide):
    mesh = plsc.VectorSubcoreMesh(core_axis_name="c", subcore_axis_name="s")
    return pl.kernel(
        _emb_body,
        out_type=jax.ShapeDtypeStruct((OUT_WROWS, 128), jnp.float32),
        mesh=mesh,
        scratch_types=[
            pltpu.VMEM((CHUNK,), jnp.int32),
            pltpu.VMEM((CHUNK,), jnp.int32),
            pltpu.VMEM((CHUNK,), jnp.int32),
            pltpu.VMEM((CHUNK,), jnp.int32),
            pltpu.VMEM((CHUNK, 128), jnp.float32),
            pltpu.VMEM((CHUNK, 128), jnp.float32),
            pltpu.VMEM((CHUNK // 4, 128), jnp.float32),
            pltpu.VMEM((CHUNK // 4, 128), jnp.float32),
            pltpu.SemaphoreType.DMA,
            pltpu.SemaphoreType.DMA,
            pltpu.SemaphoreType.DMA,
            pltpu.SemaphoreType.DMA,
        ],
        compiler_params=pltpu.CompilerParams(use_tc_tiling_on_sc=True,
                                             needs_layout_passes=False),
    )(x_flat, table_wide)


def kernel(x, table):
    out = _embed(x.reshape(-1), table.reshape(WIDE_ROWS, 128))
    return out.reshape(BATCH, FIELDS, EMBED_DIM)
